# baseline (speedup 1.0000x reference)
import numpy as np
import jax
import jax.numpy as jnp
from jax.experimental import pallas as pl
from jax.experimental.pallas import tpu as pltpu

F32 = jnp.float32
LEAK = 0.01
B = 4
LN = 128
GEOM = [(64, 64), (32, 32), (16, 16), (8, 8), (4, 4), (2, 2)]
MOFF = [0, 4096, 5120, 5376, 5440, 5456]
MTOT = 5464
TAPS22 = ((0, 0), (0, 1), (1, 0), (1, 1))


def _geo(l):
    H, W = GEOM[l]
    hw = H * W
    pad = max(8, -((W + 1) // -8) * 8)
    return H, W, hw, pad


def _lrelu(v):
    return jnp.where(v >= 0, v, LEAK * v)


def _stage(scr, v, ma, mb, l, zero=True):
    _, _, hw, pad = _geo(l)
    slab = 2 * pad + hw
    L = v.shape[1]
    if zero:
        scr[0:pad, :] = jnp.zeros((pad, 384), F32)
        scr[pad + hw - 1:slab, :] = jnp.zeros((pad + 1, 384), F32)
    if L < LN:
        ma = ma[:, :L]
        mb = mb[:, :L]
    scr[pad:pad + hw + 1, 0:L] = jnp.concatenate(
        [jnp.zeros((1, L), F32), v * ma], axis=0)
    scr[pad:pad + hw, L:2 * L] = v
    scr[pad - 8:pad - 1 + hw, 2 * L:3 * L] = jnp.concatenate(
        [jnp.zeros((7, L), F32), v * mb], axis=0)


def _conv(scr, getw, b, l, kl):
    _, W, hw, pad = _geo(l)
    acc = b
    for j, dy in enumerate((-1, 0, 1)):
        t = scr[pad + dy * W:pad + dy * W + hw, 0:kl]
        acc = acc + jnp.dot(t, getw(j), preferred_element_type=F32)
    return acc


def _pool_strided(s3, c, pwl, bp, l):
    H, W, hw, _ = _geo(l)
    hq = hw // 4
    s3[:, :, :] = c.reshape(H, W, LN)
    acc = bp
    for ky in (0, 1):
        t0 = s3[pl.ds(ky, H // 2, 2), pl.ds(0, W // 2, 2), :].reshape(hq, LN)
        t1 = s3[pl.ds(ky, H // 2, 2), pl.ds(1, W // 2, 2), :].reshape(hq, LN)
        tp = jnp.concatenate([t0, t1], axis=1)
        acc = acc + jnp.dot(tp, pwl[ky], preferred_element_type=F32)
    return acc


def _pool_sel(d_ref, c, pwl, bp):
    acc = bp
    for t, (ky, kx) in enumerate(TAPS22):
        dt = jnp.dot(d_ref[t], c, preferred_element_type=F32)
        acc = acc + jnp.dot(dt, pwl[ky, 128 * kx:128 * (kx + 1), :],
                            preferred_element_type=F32)
    return acc


def _up_strided(s3, s, uw, bu, lo):
    Hl, Wl, hwl, _ = _geo(lo + 1)
    sall = jnp.dot(s, uw, preferred_element_type=F32)
    for t, (ky, kx) in enumerate(TAPS22):
        st = sall[:, 128 * t:128 * (t + 1)].reshape(Hl, Wl, LN)
        s3[pl.ds(ky, Hl, 2), pl.ds(kx, Wl, 2), :] = st
    return s3[:, :, :].reshape(4 * hwl, LN) + bu


def _up_sel(e_ref, s, uw, bu):
    sall = jnp.dot(s, uw, preferred_element_type=F32)
    u = bu
    for t in range(4):
        u = u + jnp.dot(e_ref[t], sall[:, 128 * t:128 * (t + 1)],
                        preferred_element_type=F32)
    return u


def _net_body(x_ref, cw, hwt, twt, pwp, pwu, bias, mm, d8, d44, e8, e44,
              o_ref, scr, s3d1, s3d2, s3d3):
    s3ds = (s3d1, s3d2, s3d3)

    def masks(l):
        o = MOFF[l]
        hw = _geo(l)[2]
        return mm[0, o:o + hw, :], mm[1, o:o + hw, :]

    def brow(i, co=LN):
        return bias[i:i + 1, 0:co]

    def dconv(v, widx, b1i, b2i, l):
        ma, mb = masks(l)
        _stage(scr, v, ma, mb, l)
        h1 = _lrelu(_conv(scr, lambda j: cw[widx, j], brow(b1i), l, 384))
        _stage(scr, h1, ma, mb, l, zero=False)
        return _lrelu(_conv(scr, lambda j: cw[widx + 1, j], brow(b2i), l, 384))

    x = x_ref[0]
    ma1, mb1 = masks(0)
    _stage(scr, x, ma1, mb1, 0)
    xh = _lrelu(_conv(scr, lambda j: hwt[j], brow(0), 0, 96))

    cs, ps = [], []
    cur = xh
    for i in range(5):
        c = dconv(cur, 2 * i, 1 + 2 * i, 2 + 2 * i, i)
        if i <= 2:
            p = _pool_strided(s3ds[i], c, pwp[i], brow(13 + i), i)
        else:
            p = _pool_sel(d8 if i == 3 else d44, c, pwp[i], brow(13 + i))
        cs.append(c)
        ps.append(p)
        cur = p

    prev = dconv(cur, 10, 11, 12, 5)

    for i4 in (4, 3, 2, 1, 0):
        s = prev + ps[i4]
        if i4 == 4:
            u = _up_sel(e44, s, pwu[i4], brow(18 + i4))
        elif i4 == 3:
            u = _up_sel(e8, s, pwu[i4], brow(18 + i4))
        else:
            u = _up_strided(s3ds[i4], s, pwu[i4], brow(18 + i4), i4)
        ma, mb = masks(i4)
        w1 = 12 + 3 * i4
        _stage(scr, u, ma, mb, i4)
        acc = _conv(scr, lambda j: cw[w1, j], brow(23 + 2 * i4), i4, 384)
        _stage(scr, cs[i4], ma, mb, i4, zero=False)
        acc = acc + _conv(scr, lambda j: cw[w1 + 1, j],
                          jnp.zeros((1, LN), F32), i4, 384)
        h1 = _lrelu(acc)
        _stage(scr, h1, ma, mb, i4, zero=False)
        prev = _lrelu(_conv(scr, lambda j: cw[w1 + 2, j],
                            brow(24 + 2 * i4), i4, 384))

    _stage(scr, prev + xh, ma1, mb1, 0, zero=False)
    o_ref[0] = _lrelu(_conv(scr, lambda j: twt[j], brow(33, 4), 0, 384))


def _blkdiag(wt):
    return jnp.kron(jnp.eye(B, dtype=F32), wt)


def _conv_taps(w):
    co = w.shape[0]
    ci = w.shape[1] // 9
    r = w.reshape(co, 3, 3, ci)
    rows = []
    for ky in range(3):
        rows.append(jnp.concatenate(
            [_blkdiag(r[:, ky, kx, :].T) for kx in range(3)], axis=0))
    return jnp.stack(rows)


def _pool_taps(w):
    g = w.shape[0]
    r = w.reshape(g, 2, 2, g)
    return jnp.stack([
        jnp.concatenate([_blkdiag(r[:, ky, kx, :].T) for kx in (0, 1)], axis=0)
        for ky in (0, 1)])


def _up_taps(w):
    co = w.shape[0] // 4
    return jnp.concatenate(
        [_blkdiag(w[co * t:co * (t + 1), :].T) for t in range(4)], axis=1)


def _bias_row(b):
    return jnp.tile(b[:, 0], B)


def _np_masks():
    cols = []
    for (hn, wn) in GEOM:
        col = np.arange(hn * wn) % wn
        cols.append(np.stack([(col != wn - 1), (col != 0)], 0))
    m = np.concatenate(cols, axis=1).astype(np.float32)
    m = np.pad(m, ((0, 0), (0, MTOT - m.shape[1])))
    return np.broadcast_to(m[:, :, None], (2, MTOT, LN)).copy()


def _np_dec(hn, wn):
    hw = hn * wn
    d = np.zeros((4, hw // 4, hw), np.float32)
    for t, (ky, kx) in enumerate(TAPS22):
        for i in range(hn // 2):
            for j in range(wn // 2):
                d[t, i * (wn // 2) + j, (2 * i + ky) * wn + 2 * j + kx] = 1.0
    return d


_MM = _np_masks()
_D8 = _np_dec(8, 8)
_D44 = _np_dec(4, 4)
_E8 = np.ascontiguousarray(np.transpose(_D8, (0, 2, 1)))
_E44 = np.ascontiguousarray(np.transpose(_D44, (0, 2, 1)))


def kernel(x, head_w, head_b, left1_w1, left1_b1, left1_w2, left1_b2, pool1_w, pool1_b, up1_w, up1_b, right1_w1u, right1_w1c, right1_b1, right1_w2, right1_b2, left2_w1, left2_b1, left2_w2, left2_b2, pool2_w, pool2_b, up2_w, up2_b, right2_w1u, right2_w1c, right2_b1, right2_w2, right2_b2, left3_w1, left3_b1, left3_w2, left3_b2, pool3_w, pool3_b, up3_w, up3_b, right3_w1u, right3_w1c, right3_b1, right3_w2, right3_b2, left4_w1, left4_b1, left4_w2, left4_b2, pool4_w, pool4_b, up4_w, up4_b, right4_w1u, right4_w1c, right4_b1, right4_w2, right4_b2, left5_w1, left5_b1, left5_w2, left5_b2, pool5_w, pool5_b, up5_w, up5_b, right5_w1u, right5_w1c, right5_b1, right5_w2, right5_b2, rem_w1, rem_b1, rem_w2, rem_b2, tail_w, tail_b):
    N, cin, H, W = x.shape
    hw = H * W
    S = N // B

    lw = [(left1_w1, left1_w2), (left2_w1, left2_w2), (left3_w1, left3_w2),
          (left4_w1, left4_w2), (left5_w1, left5_w2)]
    rw = [(right1_w1u, right1_w1c, right1_w2),
          (right2_w1u, right2_w1c, right2_w2),
          (right3_w1u, right3_w1c, right3_w2),
          (right4_w1u, right4_w1c, right4_w2),
          (right5_w1u, right5_w1c, right5_w2)]
    convs = [w for pair in lw for w in pair] + [rem_w1, rem_w2] \
        + [w for tri in rw for w in tri]
    cw = jnp.stack([_conv_taps(w) for w in convs])
    hwt = _conv_taps(head_w)
    twt = _conv_taps(tail_w)
    pwp = jnp.stack([_pool_taps(w) for w in
                     (pool1_w, pool2_w, pool3_w, pool4_w, pool5_w)])
    pwu = jnp.stack([_up_taps(w) for w in
                     (up1_w, up2_w, up3_w, up4_w, up5_w)])

    lb = [(left1_b1, left1_b2), (left2_b1, left2_b2), (left3_b1, left3_b2),
          (left4_b1, left4_b2), (left5_b1, left5_b2)]
    rb = [(right1_b1, right1_b2), (right2_b1, right2_b2),
          (right3_b1, right3_b2), (right4_b1, right4_b2),
          (right5_b1, right5_b2)]
    rows = [_bias_row(head_b)]
    rows += [_bias_row(b) for pair in lb for b in pair]
    rows += [_bias_row(rem_b1), _bias_row(rem_b2)]
    rows += [_bias_row(b) for b in (pool1_b, pool2_b, pool3_b, pool4_b, pool5_b)]
    rows += [_bias_row(b) for b in (up1_b, up2_b, up3_b, up4_b, up5_b)]
    rows += [_bias_row(b) for pair in rb for b in pair]
    rows += [jnp.pad(_bias_row(tail_b), (0, LN - B))]
    bias = jnp.pad(jnp.stack(rows), ((0, 40 - len(rows)), (0, 0)))

    xg = x.reshape(S, B, cin, hw).transpose(0, 3, 1, 2)
    xg = jnp.pad(xg, ((0, 0), (0, 0), (0, 0), (0, 8 - cin))).reshape(S, hw, 32)

    full = lambda a: pl.BlockSpec(a.shape, lambda s: (0,) * a.ndim)
    consts = [cw, hwt, twt, pwp, pwu, bias,
              jnp.asarray(_MM), jnp.asarray(_D8), jnp.asarray(_D44),
              jnp.asarray(_E8), jnp.asarray(_E44)]
    out = pl.pallas_call(
        _net_body,
        out_shape=jax.ShapeDtypeStruct((S, hw, B), F32),
        grid=(S,),
        in_specs=[pl.BlockSpec((1, hw, 32), lambda s: (s, 0, 0))]
                 + [full(a) for a in consts],
        out_specs=pl.BlockSpec((1, hw, B), lambda s: (s, 0, 0)),
        scratch_shapes=[pltpu.VMEM((4240, 384), F32),
                        pltpu.VMEM((64, 64, LN), F32),
                        pltpu.VMEM((32, 32, LN), F32),
                        pltpu.VMEM((16, 16, LN), F32)],
        compiler_params=pltpu.CompilerParams(
            dimension_semantics=("parallel",)),
    )(xg, *consts)
    return out.transpose(0, 2, 1).reshape(N, 1, H, W)


# speedup vs baseline: 1.0034x; 1.0034x over previous
import numpy as np
import jax
import jax.numpy as jnp
from jax.experimental import pallas as pl
from jax.experimental.pallas import tpu as pltpu

F32 = jnp.float32
LEAK = 0.01
B = 4
LN = 128
GEOM = [(64, 64), (32, 32), (16, 16), (8, 8), (4, 4), (2, 2)]
MOFF = [0, 4096, 5120, 5376, 5440, 5456]
MTOT = 5464
TAPS22 = ((0, 0), (0, 1), (1, 0), (1, 1))


def _geo(l):
    H, W = GEOM[l]
    hw = H * W
    pad = max(8, -((W + 1) // -8) * 8)
    return H, W, hw, pad


def _lrelu(v):
    return jnp.where(v >= 0, v, LEAK * v)


def _stage(scr, v, ma, mb, l, zero=True):
    _, _, hw, pad = _geo(l)
    slab = 2 * pad + hw
    L = v.shape[1]
    if zero:
        scr[0:pad + 1, :] = jnp.zeros((pad + 1, 384), F32)
        scr[pad + hw - 1:slab, :] = jnp.zeros((pad + 1, 384), F32)
    if L < LN:
        ma = ma[:, :L]
        mb = mb[:, :L]
    scr[pad + 1:pad + 1 + hw, 0:L] = v * ma
    scr[pad:pad + hw, L:2 * L] = v
    scr[pad - 1:pad - 1 + hw, 2 * L:3 * L] = v * mb


def _conv(scr, getw, b, l, kl):
    _, W, hw, pad = _geo(l)
    acc = b
    for j, dy in enumerate((-1, 0, 1)):
        t = scr[pad + dy * W:pad + dy * W + hw, 0:kl]
        acc = acc + jnp.dot(t, getw(j), preferred_element_type=F32)
    return acc


def _pool_strided(s3, c, pwl, bp, l):
    H, W, hw, _ = _geo(l)
    hq = hw // 4
    s3[:, :, :] = c.reshape(H, W, LN)
    acc = bp
    for ky in (0, 1):
        t0 = s3[pl.ds(ky, H // 2, 2), pl.ds(0, W // 2, 2), :].reshape(hq, LN)
        t1 = s3[pl.ds(ky, H // 2, 2), pl.ds(1, W // 2, 2), :].reshape(hq, LN)
        tp = jnp.concatenate([t0, t1], axis=1)
        acc = acc + jnp.dot(tp, pwl[ky], preferred_element_type=F32)
    return acc


def _pool_sel(d_ref, c, pwl, bp):
    acc = bp
    for t, (ky, kx) in enumerate(TAPS22):
        dt = jnp.dot(d_ref[t], c, preferred_element_type=F32)
        acc = acc + jnp.dot(dt, pwl[ky, 128 * kx:128 * (kx + 1), :],
                            preferred_element_type=F32)
    return acc


def _up_strided(s3, s, uw, bu, lo):
    Hl, Wl, hwl, _ = _geo(lo + 1)
    sall = jnp.dot(s, uw, preferred_element_type=F32)
    for t, (ky, kx) in enumerate(TAPS22):
        st = sall[:, 128 * t:128 * (t + 1)].reshape(Hl, Wl, LN)
        s3[pl.ds(ky, Hl, 2), pl.ds(kx, Wl, 2), :] = st
    return s3[:, :, :].reshape(4 * hwl, LN) + bu


def _up_sel(e_ref, s, uw, bu):
    sall = jnp.dot(s, uw, preferred_element_type=F32)
    u = bu
    for t in range(4):
        u = u + jnp.dot(e_ref[t], sall[:, 128 * t:128 * (t + 1)],
                        preferred_element_type=F32)
    return u


def _net_body(x_ref, cw, hwt, twt, pwp, pwu, bias, mm, d8, d44, e8, e44,
              o_ref, scr, s3d1, s3d2, s3d3):
    s3ds = (s3d1, s3d2, s3d3)

    def masks(l):
        o = MOFF[l]
        hw = _geo(l)[2]
        return mm[0, o:o + hw, :], mm[1, o:o + hw, :]

    def brow(i, co=LN):
        return bias[i:i + 1, 0:co]

    def dconv(v, widx, b1i, b2i, l):
        ma, mb = masks(l)
        _stage(scr, v, ma, mb, l)
        h1 = _lrelu(_conv(scr, lambda j: cw[widx, j], brow(b1i), l, 384))
        _stage(scr, h1, ma, mb, l, zero=False)
        return _lrelu(_conv(scr, lambda j: cw[widx + 1, j], brow(b2i), l, 384))

    x = x_ref[0]
    ma1, mb1 = masks(0)
    _stage(scr, x, ma1, mb1, 0)
    xh = _lrelu(_conv(scr, lambda j: hwt[j], brow(0), 0, 96))

    cs, ps = [], []
    cur = xh
    for i in range(5):
        c = dconv(cur, 2 * i, 1 + 2 * i, 2 + 2 * i, i)
        if i <= 2:
            p = _pool_strided(s3ds[i], c, pwp[i], brow(13 + i), i)
        else:
            p = _pool_sel(d8 if i == 3 else d44, c, pwp[i], brow(13 + i))
        cs.append(c)
        ps.append(p)
        cur = p

    prev = dconv(cur, 10, 11, 12, 5)

    for i4 in (4, 3, 2, 1, 0):
        s = prev + ps[i4]
        if i4 == 4:
            u = _up_sel(e44, s, pwu[i4], brow(18 + i4))
        elif i4 == 3:
            u = _up_sel(e8, s, pwu[i4], brow(18 + i4))
        else:
            u = _up_strided(s3ds[i4], s, pwu[i4], brow(18 + i4), i4)
        ma, mb = masks(i4)
        w1 = 12 + 3 * i4
        _stage(scr, u, ma, mb, i4)
        acc = _conv(scr, lambda j: cw[w1, j], brow(23 + 2 * i4), i4, 384)
        _stage(scr, cs[i4], ma, mb, i4, zero=False)
        acc = acc + _conv(scr, lambda j: cw[w1 + 1, j],
                          jnp.zeros((1, LN), F32), i4, 384)
        h1 = _lrelu(acc)
        _stage(scr, h1, ma, mb, i4, zero=False)
        prev = _lrelu(_conv(scr, lambda j: cw[w1 + 2, j],
                            brow(24 + 2 * i4), i4, 384))

    _stage(scr, prev + xh, ma1, mb1, 0, zero=False)
    o_ref[0] = _lrelu(_conv(scr, lambda j: twt[j], brow(33, 4), 0, 384))


def _blkdiag(wt):
    return jnp.kron(jnp.eye(B, dtype=F32), wt)


def _conv_taps(w):
    co = w.shape[0]
    ci = w.shape[1] // 9
    r = w.reshape(co, 3, 3, ci)
    rows = []
    for ky in range(3):
        rows.append(jnp.concatenate(
            [_blkdiag(r[:, ky, kx, :].T) for kx in range(3)], axis=0))
    return jnp.stack(rows)


def _pool_taps(w):
    g = w.shape[0]
    r = w.reshape(g, 2, 2, g)
    return jnp.stack([
        jnp.concatenate([_blkdiag(r[:, ky, kx, :].T) for kx in (0, 1)], axis=0)
        for ky in (0, 1)])


def _up_taps(w):
    co = w.shape[0] // 4
    return jnp.concatenate(
        [_blkdiag(w[co * t:co * (t + 1), :].T) for t in range(4)], axis=1)


def _bias_row(b):
    return jnp.tile(b[:, 0], B)


def _np_masks():
    cols = []
    for (hn, wn) in GEOM:
        col = np.arange(hn * wn) % wn
        cols.append(np.stack([(col != wn - 1), (col != 0)], 0))
    m = np.concatenate(cols, axis=1).astype(np.float32)
    m = np.pad(m, ((0, 0), (0, MTOT - m.shape[1])))
    return np.broadcast_to(m[:, :, None], (2, MTOT, LN)).copy()


def _np_dec(hn, wn):
    hw = hn * wn
    d = np.zeros((4, hw // 4, hw), np.float32)
    for t, (ky, kx) in enumerate(TAPS22):
        for i in range(hn // 2):
            for j in range(wn // 2):
                d[t, i * (wn // 2) + j, (2 * i + ky) * wn + 2 * j + kx] = 1.0
    return d


_MM = _np_masks()
_D8 = _np_dec(8, 8)
_D44 = _np_dec(4, 4)
_E8 = np.ascontiguousarray(np.transpose(_D8, (0, 2, 1)))
_E44 = np.ascontiguousarray(np.transpose(_D44, (0, 2, 1)))


def kernel(x, head_w, head_b, left1_w1, left1_b1, left1_w2, left1_b2, pool1_w, pool1_b, up1_w, up1_b, right1_w1u, right1_w1c, right1_b1, right1_w2, right1_b2, left2_w1, left2_b1, left2_w2, left2_b2, pool2_w, pool2_b, up2_w, up2_b, right2_w1u, right2_w1c, right2_b1, right2_w2, right2_b2, left3_w1, left3_b1, left3_w2, left3_b2, pool3_w, pool3_b, up3_w, up3_b, right3_w1u, right3_w1c, right3_b1, right3_w2, right3_b2, left4_w1, left4_b1, left4_w2, left4_b2, pool4_w, pool4_b, up4_w, up4_b, right4_w1u, right4_w1c, right4_b1, right4_w2, right4_b2, left5_w1, left5_b1, left5_w2, left5_b2, pool5_w, pool5_b, up5_w, up5_b, right5_w1u, right5_w1c, right5_b1, right5_w2, right5_b2, rem_w1, rem_b1, rem_w2, rem_b2, tail_w, tail_b):
    N, cin, H, W = x.shape
    hw = H * W
    S = N // B

    lw = [(left1_w1, left1_w2), (left2_w1, left2_w2), (left3_w1, left3_w2),
          (left4_w1, left4_w2), (left5_w1, left5_w2)]
    rw = [(right1_w1u, right1_w1c, right1_w2),
          (right2_w1u, right2_w1c, right2_w2),
          (right3_w1u, right3_w1c, right3_w2),
          (right4_w1u, right4_w1c, right4_w2),
          (right5_w1u, right5_w1c, right5_w2)]
    convs = [w for pair in lw for w in pair] + [rem_w1, rem_w2] \
        + [w for tri in rw for w in tri]
    cw = jnp.stack([_conv_taps(w) for w in convs])
    hwt = _conv_taps(head_w)
    twt = _conv_taps(tail_w)
    pwp = jnp.stack([_pool_taps(w) for w in
                     (pool1_w, pool2_w, pool3_w, pool4_w, pool5_w)])
    pwu = jnp.stack([_up_taps(w) for w in
                     (up1_w, up2_w, up3_w, up4_w, up5_w)])

    lb = [(left1_b1, left1_b2), (left2_b1, left2_b2), (left3_b1, left3_b2),
          (left4_b1, left4_b2), (left5_b1, left5_b2)]
    rb = [(right1_b1, right1_b2), (right2_b1, right2_b2),
          (right3_b1, right3_b2), (right4_b1, right4_b2),
          (right5_b1, right5_b2)]
    rows = [_bias_row(head_b)]
    rows += [_bias_row(b) for pair in lb for b in pair]
    rows += [_bias_row(rem_b1), _bias_row(rem_b2)]
    rows += [_bias_row(b) for b in (pool1_b, pool2_b, pool3_b, pool4_b, pool5_b)]
    rows += [_bias_row(b) for b in (up1_b, up2_b, up3_b, up4_b, up5_b)]
    rows += [_bias_row(b) for pair in rb for b in pair]
    rows += [jnp.pad(_bias_row(tail_b), (0, LN - B))]
    bias = jnp.pad(jnp.stack(rows), ((0, 40 - len(rows)), (0, 0)))

    xg = x.reshape(S, B, cin, hw).transpose(0, 3, 1, 2)
    xg = jnp.pad(xg, ((0, 0), (0, 0), (0, 0), (0, 8 - cin))).reshape(S, hw, 32)

    full = lambda a: pl.BlockSpec(a.shape, lambda s: (0,) * a.ndim)
    consts = [cw, hwt, twt, pwp, pwu, bias,
              jnp.asarray(_MM), jnp.asarray(_D8), jnp.asarray(_D44),
              jnp.asarray(_E8), jnp.asarray(_E44)]
    out = pl.pallas_call(
        _net_body,
        out_shape=jax.ShapeDtypeStruct((S, hw, B), F32),
        grid=(S,),
        in_specs=[pl.BlockSpec((1, hw, 32), lambda s: (s, 0, 0))]
                 + [full(a) for a in consts],
        out_specs=pl.BlockSpec((1, hw, B), lambda s: (s, 0, 0)),
        scratch_shapes=[pltpu.VMEM((4240, 384), F32),
                        pltpu.VMEM((64, 64, LN), F32),
                        pltpu.VMEM((32, 32, LN), F32),
                        pltpu.VMEM((16, 16, LN), F32)],
        compiler_params=pltpu.CompilerParams(
            dimension_semantics=("parallel",)),
    )(xg, *consts)
    return out.transpose(0, 2, 1).reshape(N, 1, H, W)
